# Initial kernel scaffold; baseline (speedup 1.0000x reference)
import jax
import jax.numpy as jnp
from jax.experimental import pallas as pl


def kernel(x, edge_index, edge_attr, W1, b1, W2, b2, root, bias, Wf, bf):
    raise NotImplementedError("write your pallas kernel here")


# speedup vs baseline: 4.3705x; 4.3705x over previous
import functools

import jax
import jax.numpy as jnp
from jax import lax
from jax.experimental import pallas as pl
from jax.experimental.pallas import tpu as pltpu
from jax.experimental.pallas import tpu_sc as plsc

N_NODES = 10000
N_EDGES = 160000
IN_CH = 128
EDGE_CH = 16
OUT_CH = 4
HID = 8

N_PAD = 10240
E_PAD = 163840
EBLK = 512
NBLK = E_PAD // EBLK
NW = 32
BLK_PER_W = NBLK // NW
ROWW = 40
MSGW = 8
STRIPE = N_PAD // 16


def _precompute_body(x_ref, w2x_ref, root_ref, t40_ref, xr_ref):
    xb = x_ref[...]
    t40_ref[...] = jnp.dot(xb, w2x_ref[...], preferred_element_type=jnp.float32)
    xr_ref[...] = jnp.dot(xb, root_ref[...], preferred_element_type=jnp.float32)


def _edge_mlp_body(ea_ref, w1t_ref, b1b_ref, h_ref):
    h = jnp.dot(ea_ref[...], w1t_ref[...], preferred_element_type=jnp.float32)
    h_ref[...] = jnp.maximum(h + b1b_ref[...][0:1, :], 0.0)


def kernel(x, edge_index, edge_attr, W1, b1, W2, b2, root, bias, Wf, bf):
    f32 = jnp.float32
    i32 = jnp.int32

    src = edge_index[0].astype(i32)
    dst = edge_index[1].astype(i32)
    pad_e = E_PAD - N_EDGES
    src_p = jnp.concatenate([src, jnp.zeros((pad_e,), i32)]).reshape(E_PAD // 128, 128)
    dst_p = jnp.concatenate([dst, jnp.full((pad_e,), N_NODES, i32)]).reshape(E_PAD // 128, 128)
    ea_p = jnp.concatenate(
        [edge_attr.astype(f32), jnp.zeros((pad_e, EDGE_CH), f32)], axis=0)
    x_p = jnp.concatenate(
        [x.astype(f32), jnp.zeros((N_PAD - N_NODES, IN_CH), f32)], axis=0)
    w2r = W2.astype(f32).reshape(IN_CH, OUT_CH, HID)
    b2r = b2.astype(f32).reshape(IN_CH, OUT_CH)
    w2x = jnp.concatenate(
        [w2r, b2r[:, :, None], jnp.zeros((IN_CH, OUT_CH, 1), f32)],
        axis=2).reshape(IN_CH, OUT_CH * 10)
    w1t = W1.astype(f32).T
    b1b = jnp.broadcast_to(b1.astype(f32), (8, HID))
    biasb = jnp.broadcast_to(bias.astype(f32), (8, OUT_CH))
    wft = Wf.astype(f32).T
    bfb = jnp.broadcast_to(bf.astype(f32), (8, OUT_CH))
    zrows = jnp.zeros((N_PAD, MSGW), f32)

    nb = N_PAD // 1024
    t40, xr = pl.pallas_call(
        _precompute_body,
        grid=(nb,),
        in_specs=[
            pl.BlockSpec((1024, IN_CH), lambda i: (i, 0)),
            pl.BlockSpec((IN_CH, ROWW), lambda i: (0, 0)),
            pl.BlockSpec((IN_CH, OUT_CH), lambda i: (0, 0)),
        ],
        out_specs=[
            pl.BlockSpec((1024, ROWW), lambda i: (i, 0)),
            pl.BlockSpec((1024, OUT_CH), lambda i: (i, 0)),
        ],
        out_shape=[
            jax.ShapeDtypeStruct((N_PAD, ROWW), f32),
            jax.ShapeDtypeStruct((N_PAD, OUT_CH), f32),
        ],
    )(x_p, w2x, root.astype(f32))

    eb = E_PAD // 2048
    h = pl.pallas_call(
        _edge_mlp_body,
        grid=(eb,),
        in_specs=[
            pl.BlockSpec((2048, EDGE_CH), lambda i: (i, 0)),
            pl.BlockSpec((EDGE_CH, HID), lambda i: (0, 0)),
            pl.BlockSpec((8, HID), lambda i: (0, 0)),
        ],
        out_specs=pl.BlockSpec((2048, HID), lambda i: (i, 0)),
        out_shape=jax.ShapeDtypeStruct((E_PAD, HID), f32),
    )(ea_p, w1t, b1b)

    mesh = plsc.VectorSubcoreMesh(core_axis_name="c", subcore_axis_name="s")

    def sc_body(t40_hbm, h_hbm, src_hbm, dst_hbm, z_hbm, out_hbm,
                srcv, dstv, hv, tg, msgv, accs, gsem):
        c = lax.axis_index("c")
        s = lax.axis_index("s")
        wid = s * 2 + c
        stripe0 = s * STRIPE
        iota = lax.iota(i32, 16)

        pltpu.sync_copy(z_hbm.at[pl.ds(stripe0, STRIPE)],
                        accs.at[pl.ds(stripe0, STRIPE)])
        def initg(g, carry):
            rows = iota + g * 16
            plsc.store_scatter(msgv, [rows, jnp.full((16,), 4, i32)],
                               jnp.full((16,), 1.0, f32))
            for col in (5, 6, 7):
                plsc.store_scatter(msgv, [rows, jnp.full((16,), col, i32)],
                                   jnp.zeros((16,), f32))
            return carry
        lax.fori_loop(0, EBLK // 16, initg, 0)
        plsc.subcore_barrier()

        def block_body(t, carry):
            b = wid * BLK_PER_W + t
            pltpu.sync_copy(src_hbm.at[pl.ds(b * 4, 4)], srcv)
            pltpu.sync_copy(dst_hbm.at[pl.ds(b * 4, 4)], dstv)
            pltpu.sync_copy(h_hbm.at[pl.ds(b * EBLK, EBLK)], hv)
            cps = [pltpu.async_copy(t40_hbm.at[srcv.at[j]],
                                    tg.at[pl.ds(j * 128, 128)], gsem)
                   for j in range(4)]
            for cp in cps:
                cp.wait()

            def group(g, gcarry):
                rows = iota + g * 16
                hk = [plsc.load_gather(hv, [rows, jnp.full((16,), k, i32)])
                      for k in range(HID)]
                for o in range(OUT_CH):
                    acc = plsc.load_gather(
                        tg, [rows, jnp.full((16,), o * 10 + 8, i32)])
                    for k in range(HID):
                        acc = acc + hk[k] * plsc.load_gather(
                            tg, [rows, jnp.full((16,), o * 10 + k, i32)])
                    plsc.store_scatter(msgv, [rows, jnp.full((16,), o, i32)], acc)
                return gcarry
            lax.fori_loop(0, EBLK // 16, group, 0)

            for j in range(4):
                pltpu.sync_copy(msgv.at[pl.ds(j * 128, 128)],
                                accs.at[dstv.at[j]], add=True)
            return carry
        lax.fori_loop(0, BLK_PER_W, block_body, 0)

        plsc.subcore_barrier()
        pltpu.sync_copy(accs.at[pl.ds(stripe0, STRIPE)],
                        out_hbm.at[c, pl.ds(stripe0, STRIPE)])

    sc_call = pl.kernel(
        sc_body,
        out_type=jax.ShapeDtypeStruct((2, N_PAD, MSGW), f32),
        mesh=mesh,
        compiler_params=pltpu.CompilerParams(
            needs_layout_passes=False, use_tc_tiling_on_sc=False),
        scratch_types=[
            pltpu.VMEM((4, 128), i32),
            pltpu.VMEM((4, 128), i32),
            pltpu.VMEM((EBLK, HID), f32),
            pltpu.VMEM((EBLK, ROWW), f32),
            pltpu.VMEM((EBLK, MSGW), f32),
            pltpu.VMEM_SHARED((N_PAD, MSGW), f32),
            pltpu.SemaphoreType.DMA,
        ],
    )
    parts = sc_call(t40, h, src_p, dst_p, zrows)

    def combine_body(p0_ref, p1_ref, xr_ref, biasb_ref, wft_ref, bfb_ref, out_ref):
        p0 = p0_ref[...]
        p1 = p1_ref[...]
        s4 = p0[:, 0:OUT_CH] + p1[:, 0:OUT_CH]
        cnt = p0[:, OUT_CH:OUT_CH + 1] + p1[:, OUT_CH:OUT_CH + 1]
        agg = s4 / jnp.maximum(cnt, 1.0)
        outn = jnp.maximum(agg + xr_ref[...] + biasb_ref[...][0:1, :], 0.0)
        rowid = lax.broadcasted_iota(i32, (N_PAD, 1), 0)
        outn = jnp.where(rowid < N_NODES, outn, 0.0)
        y = jnp.dot(outn, wft_ref[...], preferred_element_type=f32)
        pooled = jnp.sum(y, axis=0, keepdims=True) / float(N_NODES)
        out_ref[...] = pooled + bfb_ref[...][0:1, :]

    out = pl.pallas_call(
        combine_body,
        out_shape=jax.ShapeDtypeStruct((1, OUT_CH), f32),
    )(parts[0], parts[1], xr, biasb, wft, bfb)
    return out
